# baseline (speedup 1.0000x reference)
import functools

import jax
import jax.numpy as jnp
from jax.experimental import pallas as pl
from jax.experimental.pallas import tpu as pltpu

_BLK = 128
_IOU_THR = 0.5
_MAX_DET = 100


def _nms_mask_kernel(b_ref, bT_ref, sup_ref, s_blk):
    npad = b_ref.shape[0]
    nblk = npad // _BLK

    sup_ref[...] = jnp.zeros((1, npad), jnp.int32)

    lane_b = jax.lax.broadcasted_iota(jnp.int32, (1, _BLK), 1)

    def blk_body(i, carry):
        start = i * _BLK
        blk = b_ref[pl.ds(start, _BLK), :]
        x1b = blk[:, 0:1]
        y1b = blk[:, 1:2]
        x2b = blk[:, 2:3]
        y2b = blk[:, 3:4]
        area_b = (x2b - x1b) * (y2b - y1b)

        bt = bT_ref[:, pl.ds(start, _BLK)]
        x1r = bt[0:1, :]
        y1r = bt[1:2, :]
        x2r = bt[2:3, :]
        y2r = bt[3:4, :]
        area_r = (x2r - x1r) * (y2r - y1r)
        wb = jnp.maximum(jnp.minimum(x2b, x2r) - jnp.maximum(x1b, x1r), 0.0)
        hb = jnp.maximum(jnp.minimum(y2b, y2r) - jnp.maximum(y1b, y1r), 0.0)
        interb = wb * hb
        ioub = interb / (area_b + area_r - interb + 1e-9)
        overb = ioub > _IOU_THR
        s_blk[...] = overb.astype(jnp.int32)

        supb0 = sup_ref[:, pl.ds(start, _BLK)]
        row_i = jax.lax.broadcasted_iota(jnp.int32, (_BLK, _BLK), 0)
        col_i = jax.lax.broadcasted_iota(jnp.int32, (_BLK, _BLK), 1)
        act0 = jnp.any(overb & (row_i > col_i), axis=0, keepdims=True)
        act0 = (act0 & (supb0 == 0)).astype(jnp.int32)

        def cond(c):
            _, a = c
            return jnp.max(a) > 0

        def body(c):
            sb, a = c
            j = jnp.min(jnp.where(a > 0, lane_b, _BLK))
            row = s_blk[pl.ds(j, 1), :]
            sb2 = sb | (((lane_b > j) & (row > 0)).astype(jnp.int32))
            a2 = a & (1 - sb2) & ((lane_b != j).astype(jnp.int32))
            return sb2, a2

        supb, _ = jax.lax.while_loop(cond, body, (supb0, act0))
        sup_ref[:, pl.ds(start, _BLK)] = supb

        kept = (supb == 0).astype(jnp.float32)

        def col_body(k, c):
            cs = k * _BLK
            bt_c = bT_ref[:, pl.ds(cs, _BLK)]
            x1c = bt_c[0:1, :]
            y1c = bt_c[1:2, :]
            x2c = bt_c[2:3, :]
            y2c = bt_c[3:4, :]
            area_c = (x2c - x1c) * (y2c - y1c)
            wc = jnp.maximum(jnp.minimum(x2b, x2c) - jnp.maximum(x1b, x1c), 0.0)
            hc = jnp.maximum(jnp.minimum(y2b, y2c) - jnp.maximum(y1b, y1c), 0.0)
            ic = wc * hc
            iouc = ic / (area_b + area_c - ic + 1e-9)
            overc = (iouc > _IOU_THR).astype(jnp.float32)
            contrib = jnp.dot(kept, overc, preferred_element_type=jnp.float32)
            cur = sup_ref[:, pl.ds(cs, _BLK)]
            sup_ref[:, pl.ds(cs, _BLK)] = cur | (contrib > 0.0).astype(jnp.int32)
            return c

        jax.lax.fori_loop(i + 1, nblk, col_body, 0)
        return carry

    jax.lax.fori_loop(0, nblk, blk_body, 0)


@functools.partial(jax.jit, static_argnames=())
def kernel(boxes, scores, class_idxs):
    n = boxes.shape[0]
    npad = ((n + _BLK - 1) // _BLK) * _BLK

    max_coord = jnp.max(boxes) + 1.0
    offsets = class_idxs.astype(boxes.dtype) * max_coord
    boxes_for_nms = boxes + offsets[:, None]

    order = jnp.argsort(-scores)
    b_sorted = boxes_for_nms[order]
    b_pad = jnp.zeros((npad, 4), jnp.float32).at[:n, :].set(b_sorted)
    bT_pad = b_pad.T

    sup = pl.pallas_call(
        _nms_mask_kernel,
        out_shape=jax.ShapeDtypeStruct((1, npad), jnp.int32),
        scratch_shapes=[pltpu.VMEM((_BLK, _BLK), jnp.int32)],
    )(b_pad, bT_pad)

    suppressed = sup[0, :n] > 0
    kept_scores = jnp.where(suppressed, -jnp.inf, scores[order])
    _, topk_idx = jax.lax.top_k(kept_scores, _MAX_DET)
    final_idx = order[topk_idx]
    return boxes[final_idx], scores[final_idx], class_idxs[final_idx]


# speedup vs baseline: 1.5570x; 1.5570x over previous
import functools

import jax
import jax.numpy as jnp
from jax.experimental import pallas as pl
from jax.experimental.pallas import tpu as pltpu

_BLK = 128
_CHUNK = 1024
_IOU_THR = 0.5
_MAX_DET = 100


def _nms_mask_kernel(b_ref, bT_ref, sup_ref, s_blk):
    npad = b_ref.shape[0]
    nblk = npad // _BLK

    sup_ref[...] = jnp.zeros((1, npad), jnp.int32)

    lane_b = jax.lax.broadcasted_iota(jnp.int32, (1, _BLK), 1)

    def blk_body(i, carry):
        start = i * _BLK
        blk = b_ref[pl.ds(start, _BLK), :]
        x1b = blk[:, 0:1]
        y1b = blk[:, 1:2]
        x2b = blk[:, 2:3]
        y2b = blk[:, 3:4]
        area_b = (x2b - x1b) * (y2b - y1b)

        bt = bT_ref[:, pl.ds(start, _BLK)]
        x1r = bt[0:1, :]
        y1r = bt[1:2, :]
        x2r = bt[2:3, :]
        y2r = bt[3:4, :]
        area_r = (x2r - x1r) * (y2r - y1r)
        wb = jnp.maximum(jnp.minimum(x2b, x2r) - jnp.maximum(x1b, x1r), 0.0)
        hb = jnp.maximum(jnp.minimum(y2b, y2r) - jnp.maximum(y1b, y1r), 0.0)
        interb = wb * hb
        ioub = interb / (area_b + area_r - interb + 1e-9)
        overb = ioub > _IOU_THR
        s_blk[...] = overb.astype(jnp.int32)

        supb0 = sup_ref[:, pl.ds(start, _BLK)]
        row_i = jax.lax.broadcasted_iota(jnp.int32, (_BLK, _BLK), 0)
        col_i = jax.lax.broadcasted_iota(jnp.int32, (_BLK, _BLK), 1)
        act0 = jnp.any(overb & (row_i > col_i), axis=0, keepdims=True)
        act0 = (act0 & (supb0 == 0)).astype(jnp.int32)

        def cond(c):
            _, a = c
            return jnp.max(a) > 0

        def body(c):
            sb, a = c
            j = jnp.min(jnp.where(a > 0, lane_b, _BLK))
            row = s_blk[pl.ds(j, 1), :]
            sb2 = sb | (((lane_b > j) & (row > 0)).astype(jnp.int32))
            a2 = a & (1 - sb2) & ((lane_b != j).astype(jnp.int32))
            return sb2, a2

        supb, _ = jax.lax.while_loop(cond, body, (supb0, act0))
        sup_ref[:, pl.ds(start, _BLK)] = supb

        kept = (supb == 0).astype(jnp.float32)
        lane_c = jax.lax.broadcasted_iota(jnp.int32, (1, _CHUNK), 1)
        nch = (npad - start - _BLK + _CHUNK - 1) // _CHUNK

        def col_body(m, c):
            cs = jnp.minimum(start + _BLK + m * _CHUNK, npad - _CHUNK)
            bt_c = bT_ref[:, pl.ds(cs, _CHUNK)]
            x1c = bt_c[0:1, :]
            y1c = bt_c[1:2, :]
            x2c = bt_c[2:3, :]
            y2c = bt_c[3:4, :]
            area_c = (x2c - x1c) * (y2c - y1c)
            wc = jnp.maximum(jnp.minimum(x2b, x2c) - jnp.maximum(x1b, x1c), 0.0)
            hc = jnp.maximum(jnp.minimum(y2b, y2c) - jnp.maximum(y1b, y1c), 0.0)
            ic = wc * hc
            iouc = ic / (area_b + area_c - ic + 1e-9)
            overc = (iouc > _IOU_THR).astype(jnp.float32)
            contrib = jnp.dot(kept, overc, preferred_element_type=jnp.float32)
            valid = (cs + lane_c) >= (start + _BLK)
            cur = sup_ref[:, pl.ds(cs, _CHUNK)]
            sup_ref[:, pl.ds(cs, _CHUNK)] = cur | (
                (contrib > 0.0) & valid
            ).astype(jnp.int32)
            return c

        jax.lax.fori_loop(0, nch, col_body, 0)
        return carry

    jax.lax.fori_loop(0, nblk, blk_body, 0)


@functools.partial(jax.jit, static_argnames=())
def kernel(boxes, scores, class_idxs):
    n = boxes.shape[0]
    npad = ((n + _BLK - 1) // _BLK) * _BLK

    max_coord = jnp.max(boxes) + 1.0
    offsets = class_idxs.astype(boxes.dtype) * max_coord
    boxes_for_nms = boxes + offsets[:, None]

    order = jnp.argsort(-scores)
    b_sorted = boxes_for_nms[order]
    b_pad = jnp.zeros((npad, 4), jnp.float32).at[:n, :].set(b_sorted)
    bT_pad = b_pad.T

    sup = pl.pallas_call(
        _nms_mask_kernel,
        out_shape=jax.ShapeDtypeStruct((1, npad), jnp.int32),
        scratch_shapes=[pltpu.VMEM((_BLK, _BLK), jnp.int32)],
    )(b_pad, bT_pad)

    suppressed = sup[0, :n] > 0
    kept_scores = jnp.where(suppressed, -jnp.inf, scores[order])
    _, topk_idx = jax.lax.top_k(kept_scores, _MAX_DET)
    final_idx = order[topk_idx]
    return boxes[final_idx], scores[final_idx], class_idxs[final_idx]


# speedup vs baseline: 2.2003x; 1.4132x over previous
import functools

import jax
import jax.numpy as jnp
from jax.experimental import pallas as pl
from jax.experimental.pallas import tpu as pltpu

_BLK = 128
_CHUNK = 1024
_IOU_THR = 0.5
_MAX_DET = 100


def _nms_mask_kernel(b_ref, bT_ref, sup_ref, s_blk):
    npad = b_ref.shape[0]
    nblk = npad // _BLK

    sup_ref[...] = jnp.zeros((1, npad), jnp.int32)

    lane_b = jax.lax.broadcasted_iota(jnp.int32, (1, _BLK), 1)

    def blk_body(i, carry):
        start = i * _BLK
        blk = b_ref[pl.ds(start, _BLK), :]
        x1b = blk[:, 0:1]
        y1b = blk[:, 1:2]
        x2b = blk[:, 2:3]
        y2b = blk[:, 3:4]
        area_b = (x2b - x1b) * (y2b - y1b)

        bt = bT_ref[:, pl.ds(start, _BLK)]
        x1r = bt[0:1, :]
        y1r = bt[1:2, :]
        x2r = bt[2:3, :]
        y2r = bt[3:4, :]
        area_r = (x2r - x1r) * (y2r - y1r)
        wb = jnp.maximum(jnp.minimum(x2b, x2r) - jnp.maximum(x1b, x1r), 0.0)
        hb = jnp.maximum(jnp.minimum(y2b, y2r) - jnp.maximum(y1b, y1r), 0.0)
        interb = wb * hb
        ioub = interb / (area_b + area_r - interb + 1e-9)
        overb = ioub > _IOU_THR
        s_blk[...] = overb.astype(jnp.int32)

        supb0 = sup_ref[:, pl.ds(start, _BLK)]
        row_i = jax.lax.broadcasted_iota(jnp.int32, (_BLK, _BLK), 0)
        col_i = jax.lax.broadcasted_iota(jnp.int32, (_BLK, _BLK), 1)
        act0 = jnp.any(overb & (row_i > col_i), axis=0, keepdims=True)
        act0 = (act0 & (supb0 == 0)).astype(jnp.int32)

        def cond(c):
            _, a = c
            return jnp.max(a) > 0

        def body(c):
            sb, a = c
            j = jnp.min(jnp.where(a > 0, lane_b, _BLK))
            row = s_blk[pl.ds(j, 1), :]
            sb2 = sb | (((lane_b > j) & (row > 0)).astype(jnp.int32))
            a2 = a & (1 - sb2) & ((lane_b != j).astype(jnp.int32))
            return sb2, a2

        supb, _ = jax.lax.while_loop(cond, body, (supb0, act0))
        sup_ref[:, pl.ds(start, _BLK)] = supb

        kept = (supb == 0).astype(jnp.float32)
        lane_c = jax.lax.broadcasted_iota(jnp.int32, (1, _CHUNK), 1)
        nch = (npad - start - _BLK + _CHUNK - 1) // _CHUNK

        def col_body(m, c):
            cs = jnp.minimum(start + _BLK + m * _CHUNK, npad - _CHUNK)
            bt_c = bT_ref[:, pl.ds(cs, _CHUNK)]
            x1c = bt_c[0:1, :]
            y1c = bt_c[1:2, :]
            x2c = bt_c[2:3, :]
            y2c = bt_c[3:4, :]
            area_c = (x2c - x1c) * (y2c - y1c)
            wc = jnp.maximum(jnp.minimum(x2b, x2c) - jnp.maximum(x1b, x1c), 0.0)
            hc = jnp.maximum(jnp.minimum(y2b, y2c) - jnp.maximum(y1b, y1c), 0.0)
            ic = wc * hc
            iouc = ic / (area_b + area_c - ic + 1e-9)
            overc = (iouc > _IOU_THR).astype(jnp.float32)
            contrib = jnp.dot(kept, overc, preferred_element_type=jnp.float32)
            valid = (cs + lane_c) >= (start + _BLK)
            cur = sup_ref[:, pl.ds(cs, _CHUNK)]
            sup_ref[:, pl.ds(cs, _CHUNK)] = cur | (
                (contrib > 0.0) & valid
            ).astype(jnp.int32)
            return c

        jax.lax.fori_loop(0, nch, col_body, 0)
        return carry

    jax.lax.fori_loop(0, nblk, blk_body, 0)


@functools.partial(jax.jit, static_argnames=())
def kernel(boxes, scores, class_idxs):
    n = boxes.shape[0]
    npad = ((n + _BLK - 1) // _BLK) * _BLK

    max_coord = jnp.max(boxes) + 1.0
    cls_f = class_idxs.astype(boxes.dtype)
    neg_s, sx1, sy1, sx2, sy2, s_cls = jax.lax.sort(
        (-scores, boxes[:, 0], boxes[:, 1], boxes[:, 2], boxes[:, 3], cls_f),
        num_keys=1,
        is_stable=True,
    )
    s_scores = -neg_s
    off = s_cls * max_coord
    b_sorted = jnp.stack([sx1 + off, sy1 + off, sx2 + off, sy2 + off], axis=1)
    b_pad = jnp.zeros((npad, 4), jnp.float32).at[:n, :].set(b_sorted)
    bT_pad = b_pad.T

    sup = pl.pallas_call(
        _nms_mask_kernel,
        out_shape=jax.ShapeDtypeStruct((1, npad), jnp.int32),
        scratch_shapes=[pltpu.VMEM((_BLK, _BLK), jnp.int32)],
    )(b_pad, bT_pad)

    suppressed = sup[0, :n] > 0
    kept_scores = jnp.where(suppressed, -jnp.inf, s_scores)
    _, topk_idx = jax.lax.top_k(kept_scores, _MAX_DET)
    out_boxes = jnp.stack([sx1, sy1, sx2, sy2], axis=1)[topk_idx]
    return (
        out_boxes,
        s_scores[topk_idx],
        s_cls[topk_idx].astype(class_idxs.dtype),
    )
